# baseline (speedup 1.0000x reference)
import functools

import jax
import jax.numpy as jnp
from jax.experimental import pallas as pl
from jax.experimental.pallas import tpu as pltpu


def _proj_kernel(seq_ref, w_ref, fts_ref):
    fts_ref[...] = jax.lax.dot_general(
        seq_ref[...], w_ref[...],
        dimension_numbers=(((1,), (1,)), ((), ())),
        preferred_element_type=jnp.float32,
    )


def _agg_kernel(fts_ref, adj_ref, bias_ref, alpha_ref, out_ref):
    acc = jax.lax.dot_general(
        adj_ref[...], fts_ref[...],
        dimension_numbers=(((1,), (0,)), ((), ())),
        preferred_element_type=jnp.float32,
    )
    acc = acc + bias_ref[...]
    out_ref[...] = jnp.where(acc >= 0, acc, alpha_ref[0, 0] * acc)


@functools.partial(jax.jit, static_argnames=("interpret",))
def _gcn(seq2d, adj2d, W, bias2d, alpha2d, interpret=False):
    n, din = seq2d.shape
    dout = W.shape[0]
    tm = 400 if n % 400 == 0 else n
    grid = (n // tm,)

    fts = pl.pallas_call(
        _proj_kernel,
        out_shape=jax.ShapeDtypeStruct((n, dout), jnp.float32),
        interpret=interpret,
    )(seq2d, W)

    out = pl.pallas_call(
        _agg_kernel,
        grid=grid,
        in_specs=[
            pl.BlockSpec((n, dout), lambda i: (0, 0)),
            pl.BlockSpec((tm, n), lambda i: (i, 0)),
            pl.BlockSpec((1, dout), lambda i: (0, 0)),
            pl.BlockSpec((1, 1), lambda i: (0, 0)),
        ],
        out_specs=pl.BlockSpec((tm, dout), lambda i: (i, 0)),
        out_shape=jax.ShapeDtypeStruct((n, dout), jnp.float32),
        compiler_params=pltpu.CompilerParams(
            dimension_semantics=("parallel",),
        ),
        interpret=interpret,
    )(fts, adj2d, bias2d, alpha2d)
    return out


def kernel(seq, adj, W, bias, alpha):
    b, n, din = seq.shape
    dout = W.shape[0]
    seq2d = seq.reshape(n, din)
    adj2d = adj.reshape(n, n)
    bias2d = bias.reshape(1, dout)
    alpha2d = alpha.reshape(1, 1)
    out = _gcn(seq2d, adj2d, W, bias2d, alpha2d)
    return out.reshape(b, n, dout)


# speedup vs baseline: 1.0452x; 1.0452x over previous
import functools

import jax
import jax.numpy as jnp
from jax.experimental import pallas as pl
from jax.experimental.pallas import tpu as pltpu


def _gcn_kernel(seq_ref, w_ref, adj_ref, bias_ref, alpha_ref, out_ref, fts_ref):
    @pl.when(pl.program_id(0) == 0)
    def _():
        fts_ref[...] = jax.lax.dot_general(
            seq_ref[...], w_ref[...],
            dimension_numbers=(((1,), (1,)), ((), ())),
            preferred_element_type=jnp.float32,
        )

    acc = jax.lax.dot_general(
        adj_ref[...], fts_ref[...],
        dimension_numbers=(((1,), (0,)), ((), ())),
        preferred_element_type=jnp.float32,
    )
    acc = acc + bias_ref[...]
    out_ref[...] = jnp.where(acc >= 0, acc, alpha_ref[0, 0] * acc)


@functools.partial(jax.jit, static_argnames=("tm", "interpret"))
def _gcn(seq2d, adj2d, W, bias2d, alpha2d, tm=400, interpret=False):
    n, din = seq2d.shape
    dout = W.shape[0]
    if n % tm != 0:
        tm = n
    grid = (n // tm,)

    out = pl.pallas_call(
        _gcn_kernel,
        grid=grid,
        in_specs=[
            pl.BlockSpec((n, din), lambda i: (0, 0)),
            pl.BlockSpec((dout, din), lambda i: (0, 0)),
            pl.BlockSpec((tm, n), lambda i: (i, 0)),
            pl.BlockSpec((1, dout), lambda i: (0, 0)),
            pl.BlockSpec((1, 1), lambda i: (0, 0)),
        ],
        out_specs=pl.BlockSpec((tm, dout), lambda i: (i, 0)),
        out_shape=jax.ShapeDtypeStruct((n, dout), jnp.float32),
        scratch_shapes=[pltpu.VMEM((n, dout), jnp.float32)],
        compiler_params=pltpu.CompilerParams(
            dimension_semantics=("arbitrary",),
        ),
        interpret=interpret,
    )(seq2d, W, adj2d, bias2d, alpha2d)
    return out


def kernel(seq, adj, W, bias, alpha):
    b, n, din = seq.shape
    dout = W.shape[0]
    seq2d = seq.reshape(n, din)
    adj2d = adj.reshape(n, n)
    bias2d = bias.reshape(1, dout)
    alpha2d = alpha.reshape(1, 1)
    out = _gcn(seq2d, adj2d, W, bias2d, alpha2d)
    return out.reshape(b, n, dout)
